# baseline (speedup 1.0000x reference)
import functools
import math

import jax
import jax.numpy as jnp
from jax import lax
from jax.experimental import pallas as pl

B, N, D, H = 256, 128, 128, 8
DH = D // H

BB_ENC = 8
BB_DEC = 64

_NEG = -1e9


def _fold_sum(x):
    s = x.shape[-1]
    while s > 1:
        h = s // 2
        x = x[..., :h] + x[..., h:s]
        s = h
    return x


def _enc_kernel(s_ref, p_ref, d_ref, Wemb_ref, bemb_ref, Wq_ref, Wk_ref,
                Wv_ref, Wo_ref, g_ref, b_ref, Wkd_ref,
                mN_ref, keys_ref, gemb_ref, pupd_ref):
    bb = s_ref.shape[0]
    x = jnp.concatenate([s_ref[...], p_ref[...], d_ref[...]], axis=-1)
    m_upd = x @ Wemb_ref[...] + bemb_ref[...]

    def split(t):
        return t.reshape(bb, N, H, DH).transpose(0, 2, 1, 3).reshape(
            bb * H, N, DH)

    q = split(m_upd @ Wq_ref[...])
    k = split(m_upd @ Wk_ref[...])
    v = split(m_upd @ Wv_ref[...])
    sc = jnp.einsum('xnd,xmd->xnm', q, k) / jnp.sqrt(float(DH))
    unnorm = jnp.exp(sc - jnp.max(sc, axis=-1, keepdims=True))
    att = unnorm / _fold_sum(unnorm)
    o = jnp.einsum('xnm,xmd->xnd', att, v).reshape(
        bb, H, N, DH).transpose(0, 2, 1, 3).reshape(bb, N, D) @ Wo_ref[...]
    r = m_upd + o
    mu = _fold_sum(r) / 128.0
    var = _fold_sum((r - mu) ** 2) / 128.0
    m_n = (r - mu) / jnp.sqrt(var + 1e-5) * g_ref[...] + b_ref[...]
    mN_ref[...] = m_n
    keys_ref[...] = m_n @ Wkd_ref[...]
    gemb_ref[...] = m_n.mean(axis=1)
    pupd_ref[...] = p_ref[...].reshape(bb, N)


def _dec_kernel(mN_ref, keys_ref, gemb_ref, pupd_ref, c_ref, Wqd_ref,
                pi_ref, cost_ref, logp_ref):
    bb = mN_ref.shape[0]
    m_n = mN_ref[...]
    keys = keys_ref[...]
    gemb = gemb_ref[...]
    pupd = pupd_ref[...]
    c = c_ref[...]
    wqd = Wqd_ref[...]
    m_n_t = jnp.transpose(m_n, (0, 2, 1))
    iota_n = lax.broadcasted_iota(jnp.int32, (bb, N), 1)
    inv_sqrt_d = 1.0 / math.sqrt(float(D))
    def step(t, carry):
        visited, cur_emb, logp, pi_acc = carry
        qv = (gemb + cur_emb) @ wqd
        scores = jnp.einsum("bd,bnd->bn", qv, keys)
        logits = 10.0 * jnp.tanh(scores * inv_sqrt_d) - pupd
        logits = jnp.where(visited > 0.5, _NEG, logits)
        mx = jnp.max(logits, axis=-1, keepdims=True)
        is_max = logits == mx
        sel = jnp.min(jnp.where(is_max, iota_n, N), axis=-1,
                      keepdims=True)
        onehot = (iota_n == sel).astype(jnp.float32)
        lse_rem = jnp.log(jnp.sum(jnp.exp(logits - mx), axis=-1,
                                  keepdims=True))
        logp = logp - lse_rem
        idx = jnp.broadcast_to(sel[:, :, None], (bb, D, 1))
        new_emb = jnp.take_along_axis(m_n_t, idx, axis=2)[:, :, 0]
        visited = jnp.maximum(visited, onehot)
        step_oh = (lax.broadcasted_iota(jnp.int32, (1, N), 1) == t)
        pi_acc = pi_acc + sel * step_oh.astype(jnp.int32)
        return visited, new_emb, logp, pi_acc

    init = (jnp.zeros((bb, N), jnp.float32),
            jnp.zeros((bb, D), jnp.float32),
            jnp.zeros((bb, 1), jnp.float32),
            jnp.zeros((bb, N), jnp.int32))
    visited, cur_emb, logp, pi_acc = lax.fori_loop(0, N, step, init)

    nxt = jnp.concatenate([pi_acc[:, 1:], jnp.zeros((bb, 1), jnp.int32)],
                          axis=1)
    pi_f = pi_acc[:, :, None]
    i_iota = lax.broadcasted_iota(jnp.int32, (bb, N, N), 2)
    j_sel = jnp.sum(jnp.where(pi_f == i_iota, nxt[:, :, None], 0),
                    axis=1)
    j_iota = lax.broadcasted_iota(jnp.int32, (bb, N, N), 2)
    cost = jnp.sum(
        jnp.sum(jnp.where(j_sel[:, :, None] == j_iota, c, 0.0), axis=2),
        axis=1, keepdims=True)
    pi_ref[...] = pi_acc
    cost_ref[...] = cost
    logp_ref[...] = logp


def _encode(s, p, d, W_emb, b_emb, Wq, Wk, Wv, Wo, gamma, beta, Wk_dec):
    x = jnp.concatenate([s, p, d], axis=-1)
    m_upd = x @ W_emb + b_emb

    def split(t):
        return t.reshape(B, N, H, DH).transpose(0, 2, 1, 3)

    q = split(m_upd @ Wq)
    k = split(m_upd @ Wk)
    v = split(m_upd @ Wv)
    att = jax.nn.softmax(
        jnp.einsum('bhnd,bhmd->bhnm', q, k) / jnp.sqrt(float(DH)), axis=-1)
    o = jnp.einsum('bhnm,bhmd->bhnd', att, v).transpose(0, 2, 1, 3).reshape(
        B, N, D) @ Wo
    r = m_upd + o
    mu = r.mean(-1, keepdims=True)
    var = ((r - mu) ** 2).mean(-1, keepdims=True)
    m_n = (r - mu) / jnp.sqrt(var + 1e-5) * gamma + beta
    return m_n, m_n @ Wk_dec, m_n.mean(axis=1), p[..., 0]


def kernel(s, p, d, c, W_emb, b_emb, Wq, Wk, Wv, Wo, gamma, beta,
           Wq_dec, Wk_dec):
    full2 = lambda i: (0, 0)
    m_n, keys, gemb, pupd = _encode(
        s, p, d, W_emb, b_emb, Wq, Wk, Wv, Wo, gamma, beta, Wk_dec)

    dec_grid = (B // BB_DEC,)
    pi, cost, logp = pl.pallas_call(
        _dec_kernel,
        grid=dec_grid,
        in_specs=[
            pl.BlockSpec((BB_DEC, N, D), lambda i: (i, 0, 0)),
            pl.BlockSpec((BB_DEC, N, D), lambda i: (i, 0, 0)),
            pl.BlockSpec((BB_DEC, D), lambda i: (i, 0)),
            pl.BlockSpec((BB_DEC, N), lambda i: (i, 0)),
            pl.BlockSpec((BB_DEC, N, N), lambda i: (i, 0, 0)),
            pl.BlockSpec((D, D), full2),
        ],
        out_specs=[
            pl.BlockSpec((BB_DEC, N), lambda i: (i, 0)),
            pl.BlockSpec((BB_DEC, 1), lambda i: (i, 0)),
            pl.BlockSpec((BB_DEC, 1), lambda i: (i, 0)),
        ],
        out_shape=[
            jax.ShapeDtypeStruct((B, N), jnp.int32),
            jax.ShapeDtypeStruct((B, 1), jnp.float32),
            jax.ShapeDtypeStruct((B, 1), jnp.float32),
        ],
    )(m_n, keys, gemb, pupd, c, Wq_dec)

    return pi, cost[:, 0], logp[:, 0]


# speedup vs baseline: 1.6870x; 1.6870x over previous
import math

import jax
import jax.numpy as jnp
from jax import lax
from jax.experimental import pallas as pl

B, N, D, H = 256, 128, 128, 8
DH = D // H

BB_DEC = 128
BB_COST = 32

_NEG = -1e9


def _dec_kernel(mN_ref, keys_ref, gemb_ref, pupd_ref, Wqd_ref,
                pi_ref, logp_ref):
    bb = mN_ref.shape[0]
    m_n = mN_ref[...]
    keys = keys_ref[...]
    gemb = gemb_ref[...]
    pupd = pupd_ref[...]
    wqd = Wqd_ref[...]
    iota_n = lax.broadcasted_iota(jnp.int32, (bb, N), 1)
    inv_sqrt_d = 1.0 / math.sqrt(float(D))

    def step(t, carry):
        visited, cur_emb, logp, pi_acc = carry
        qv = (gemb + cur_emb) @ wqd
        scores = jnp.einsum("bd,bnd->bn", qv, keys)
        logits = 10.0 * jnp.tanh(scores * inv_sqrt_d) - pupd
        logits = jnp.where(visited > 0.5, _NEG, logits)
        mx = jnp.max(logits, axis=-1, keepdims=True)
        is_max = logits == mx
        sel = jnp.min(jnp.where(is_max, iota_n, N), axis=-1,
                      keepdims=True)
        onehot = (iota_n == sel).astype(jnp.float32)
        lse_rem = jnp.log(jnp.sum(jnp.exp(logits - mx), axis=-1,
                                  keepdims=True))
        logp = logp - lse_rem
        hn = N // 2
        new_emb = (jnp.sum(onehot[:, :hn, None] * m_n[:, :hn, :], axis=1)
                   + jnp.sum(onehot[:, hn:, None] * m_n[:, hn:, :], axis=1))
        visited = jnp.maximum(visited, onehot)
        step_oh = (lax.broadcasted_iota(jnp.int32, (1, N), 1) == t)
        pi_acc = pi_acc + sel * step_oh.astype(jnp.int32)
        return visited, new_emb, logp, pi_acc

    init = (jnp.zeros((bb, N), jnp.float32),
            jnp.zeros((bb, D), jnp.float32),
            jnp.zeros((bb, 1), jnp.float32),
            jnp.zeros((bb, N), jnp.int32))
    visited, cur_emb, logp, pi_acc = lax.fori_loop(0, N, step, init)
    pi_ref[...] = pi_acc
    logp_ref[...] = logp


def _cost_kernel(pi_ref, c_ref, cost_ref):
    bb = pi_ref.shape[0]
    pi_acc = pi_ref[...]
    c = c_ref[...]
    nxt = jnp.concatenate([pi_acc[:, 1:], jnp.zeros((bb, 1), jnp.int32)],
                          axis=1)
    i_iota = lax.broadcasted_iota(jnp.int32, (bb, N, N), 2)
    j_sel = jnp.sum(jnp.where(pi_acc[:, :, None] == i_iota,
                              nxt[:, :, None], 0), axis=1)
    j_iota = lax.broadcasted_iota(jnp.int32, (bb, N, N), 2)
    cost_ref[...] = jnp.sum(
        jnp.sum(jnp.where(j_sel[:, :, None] == j_iota, c, 0.0), axis=2),
        axis=1, keepdims=True)


def _encode(s, p, d, W_emb, b_emb, Wq, Wk, Wv, Wo, gamma, beta, Wk_dec):
    x = jnp.concatenate([s, p, d], axis=-1)
    m_upd = x @ W_emb + b_emb

    def split(t):
        return t.reshape(B, N, H, DH).transpose(0, 2, 1, 3)

    q = split(m_upd @ Wq)
    k = split(m_upd @ Wk)
    v = split(m_upd @ Wv)
    att = jax.nn.softmax(
        jnp.einsum('bhnd,bhmd->bhnm', q, k) / jnp.sqrt(float(DH)), axis=-1)
    o = jnp.einsum('bhnm,bhmd->bhnd', att, v).transpose(0, 2, 1, 3).reshape(
        B, N, D) @ Wo
    r = m_upd + o
    mu = r.mean(-1, keepdims=True)
    var = ((r - mu) ** 2).mean(-1, keepdims=True)
    m_n = (r - mu) / jnp.sqrt(var + 1e-5) * gamma + beta
    return m_n, m_n @ Wk_dec, m_n.mean(axis=1), p[..., 0]


def kernel(s, p, d, c, W_emb, b_emb, Wq, Wk, Wv, Wo, gamma, beta,
           Wq_dec, Wk_dec):
    full2 = lambda i: (0, 0)
    m_n, keys, gemb, pupd = _encode(
        s, p, d, W_emb, b_emb, Wq, Wk, Wv, Wo, gamma, beta, Wk_dec)

    pi, logp = pl.pallas_call(
        _dec_kernel,
        grid=(B // BB_DEC,),
        in_specs=[
            pl.BlockSpec((BB_DEC, N, D), lambda i: (i, 0, 0)),
            pl.BlockSpec((BB_DEC, N, D), lambda i: (i, 0, 0)),
            pl.BlockSpec((BB_DEC, D), lambda i: (i, 0)),
            pl.BlockSpec((BB_DEC, N), lambda i: (i, 0)),
            pl.BlockSpec((D, D), full2),
        ],
        out_specs=[
            pl.BlockSpec((BB_DEC, N), lambda i: (i, 0)),
            pl.BlockSpec((BB_DEC, 1), lambda i: (i, 0)),
        ],
        out_shape=[
            jax.ShapeDtypeStruct((B, N), jnp.int32),
            jax.ShapeDtypeStruct((B, 1), jnp.float32),
        ],
    )(m_n, keys, gemb, pupd, Wq_dec)

    cost = pl.pallas_call(
        _cost_kernel,
        grid=(B // BB_COST,),
        in_specs=[
            pl.BlockSpec((BB_COST, N), lambda i: (i, 0)),
            pl.BlockSpec((BB_COST, N, N), lambda i: (i, 0, 0)),
        ],
        out_specs=pl.BlockSpec((BB_COST, 1), lambda i: (i, 0)),
        out_shape=jax.ShapeDtypeStruct((B, 1), jnp.float32),
    )(pi, c)

    return pi, cost[:, 0], logp[:, 0]


# speedup vs baseline: 2.0340x; 1.2057x over previous
import math

import jax
import jax.numpy as jnp
from jax import lax
from jax.experimental import pallas as pl

B, N, D, H = 256, 128, 128, 8
DH = D // H

BB_DEC = 128
BB_COST = 32

_NEG = -1e9


def _dec_kernel(mN_ref, keys_ref, gemb_ref, pupd_ref, Wqd_ref,
                pi_ref, logp_ref):
    bb = mN_ref.shape[0]
    m_n = mN_ref[...]
    keys = keys_ref[...]
    gemb = gemb_ref[...]
    pupd = pupd_ref[...]
    wqd = Wqd_ref[...]
    iota_n = lax.broadcasted_iota(jnp.int32, (bb, N), 1)
    inv_sqrt_d = 1.0 / math.sqrt(float(D))

    def step(t, carry):
        visited, cur_emb, logp, pi_acc = carry
        qv = (gemb + cur_emb) @ wqd
        scores = jnp.einsum("bd,bnd->bn", qv, keys)
        logits = 10.0 * jnp.tanh(scores * inv_sqrt_d) - pupd
        logits = jnp.where(visited > 0.5, _NEG, logits)
        mx = jnp.max(logits, axis=-1, keepdims=True)
        is_max = logits == mx
        sel = jnp.min(jnp.where(is_max, iota_n, N), axis=-1,
                      keepdims=True)
        onehot = (iota_n == sel).astype(jnp.float32)
        lse_rem = jnp.log(jnp.sum(jnp.exp(logits - mx), axis=-1,
                                  keepdims=True))
        logp = logp - lse_rem
        oh3 = (lax.broadcasted_iota(jnp.int32, (bb, N, 1), 1)
               == sel[:, :, None])
        new_emb = jnp.sum(jnp.where(oh3, m_n, 0.0), axis=1)
        visited = jnp.maximum(visited, onehot)
        step_oh = (lax.broadcasted_iota(jnp.int32, (1, N), 1) == t)
        pi_acc = pi_acc + sel * step_oh.astype(jnp.int32)
        return visited, new_emb, logp, pi_acc

    init = (jnp.zeros((bb, N), jnp.float32),
            jnp.zeros((bb, D), jnp.float32),
            jnp.zeros((bb, 1), jnp.float32),
            jnp.zeros((bb, N), jnp.int32))
    visited, cur_emb, logp, pi_acc = lax.fori_loop(0, N, step, init)
    pi_ref[...] = pi_acc
    logp_ref[...] = logp


def _cost_kernel(pi_ref, c_ref, cost_ref):
    bb = pi_ref.shape[0]
    pi_acc = pi_ref[...]
    c = c_ref[...]
    nxt = jnp.concatenate([pi_acc[:, 1:], jnp.zeros((bb, 1), jnp.int32)],
                          axis=1)
    i_iota = lax.broadcasted_iota(jnp.int32, (bb, N, N), 2)
    j_sel = jnp.sum(jnp.where(pi_acc[:, :, None] == i_iota,
                              nxt[:, :, None], 0), axis=1)
    j_iota = lax.broadcasted_iota(jnp.int32, (bb, N, N), 2)
    cost_ref[...] = jnp.sum(
        jnp.sum(jnp.where(j_sel[:, :, None] == j_iota, c, 0.0), axis=2),
        axis=1, keepdims=True)


def _encode(s, p, d, W_emb, b_emb, Wq, Wk, Wv, Wo, gamma, beta, Wk_dec):
    x = jnp.concatenate([s, p, d], axis=-1)
    m_upd = x @ W_emb + b_emb

    def split(t):
        return t.reshape(B, N, H, DH).transpose(0, 2, 1, 3)

    q = split(m_upd @ Wq)
    k = split(m_upd @ Wk)
    v = split(m_upd @ Wv)
    att = jax.nn.softmax(
        jnp.einsum('bhnd,bhmd->bhnm', q, k) / jnp.sqrt(float(DH)), axis=-1)
    o = jnp.einsum('bhnm,bhmd->bhnd', att, v).transpose(0, 2, 1, 3).reshape(
        B, N, D) @ Wo
    r = m_upd + o
    mu = r.mean(-1, keepdims=True)
    var = ((r - mu) ** 2).mean(-1, keepdims=True)
    m_n = (r - mu) / jnp.sqrt(var + 1e-5) * gamma + beta
    return m_n, m_n @ Wk_dec, m_n.mean(axis=1), p[..., 0]


def kernel(s, p, d, c, W_emb, b_emb, Wq, Wk, Wv, Wo, gamma, beta,
           Wq_dec, Wk_dec):
    full2 = lambda i: (0, 0)
    m_n, keys, gemb, pupd = _encode(
        s, p, d, W_emb, b_emb, Wq, Wk, Wv, Wo, gamma, beta, Wk_dec)

    pi, logp = pl.pallas_call(
        _dec_kernel,
        grid=(B // BB_DEC,),
        in_specs=[
            pl.BlockSpec((BB_DEC, N, D), lambda i: (i, 0, 0)),
            pl.BlockSpec((BB_DEC, N, D), lambda i: (i, 0, 0)),
            pl.BlockSpec((BB_DEC, D), lambda i: (i, 0)),
            pl.BlockSpec((BB_DEC, N), lambda i: (i, 0)),
            pl.BlockSpec((D, D), full2),
        ],
        out_specs=[
            pl.BlockSpec((BB_DEC, N), lambda i: (i, 0)),
            pl.BlockSpec((BB_DEC, 1), lambda i: (i, 0)),
        ],
        out_shape=[
            jax.ShapeDtypeStruct((B, N), jnp.int32),
            jax.ShapeDtypeStruct((B, 1), jnp.float32),
        ],
    )(m_n, keys, gemb, pupd, Wq_dec)

    cost = pl.pallas_call(
        _cost_kernel,
        grid=(B // BB_COST,),
        in_specs=[
            pl.BlockSpec((BB_COST, N), lambda i: (i, 0)),
            pl.BlockSpec((BB_COST, N, N), lambda i: (i, 0, 0)),
        ],
        out_specs=pl.BlockSpec((BB_COST, 1), lambda i: (i, 0)),
        out_shape=jax.ShapeDtypeStruct((B, 1), jnp.float32),
    )(pi, c)

    return pi, cost[:, 0], logp[:, 0]
